# baseline (speedup 1.0000x reference)
import functools

import jax
import jax.numpy as jnp
from jax import lax
from jax.experimental import pallas as pl
from jax.experimental.pallas import tpu as pltpu
from jax.experimental.pallas import tpu_sc as plsc

_N = 10000
_E = 160000
_CH = 128
_NCH = _E // _CH
_NC, _NS = 2, 16
_NW = _NC * _NS
_ITERS = -(-_NCH // _NW)
_ZR = 40
_TROWS = 640


def _sc_agg(D, nslab, gather):
    mesh = plsc.VectorSubcoreMesh(core_axis_name="c", subcore_axis_name="s")
    out_type = [jax.ShapeDtypeStruct((_NC, _N, D), jnp.float32)
                for _ in range(nslab)]
    scratch = [
        pltpu.VMEM_SHARED((_N, D), jnp.float32),
        pltpu.VMEM((_ZR, D), jnp.float32),
        pltpu.VMEM((_CH, D), jnp.float32),
        pltpu.VMEM((_CH, D), jnp.float32),
        pltpu.VMEM((_ITERS, _CH), jnp.int32),
        pltpu.VMEM((_ITERS, _CH), jnp.int32),
        pltpu.SemaphoreType.DMA,
        pltpu.SemaphoreType.DMA,
        pltpu.SemaphoreType.DMA,
        pltpu.SemaphoreType.DMA,
    ]

    def body(*refs):
        n_in = (3 if gather else 2) + nslab
        ins, outs = refs[:n_in], refs[n_in:n_in + nslab]
        (acc, zbuf, rows0, rows1, srci, dsti,
         gsem0, gsem1, ssem0, ssem1) = refs[n_in + nslab:]
        if gather:
            src_h, dst_h, fill_h = ins[0], ins[1], ins[2]
            tbls = ins[3:]
        else:
            dst_h, fill_h, ones_h = ins[0], ins[1], ins[2]
            tbls = ()

        c = lax.axis_index("c")
        t = lax.axis_index("s")
        w = t * _NC + c
        c0 = _ITERS * w
        nv = lax.min(_ITERS, _NCH - c0)

        pltpu.sync_copy(fill_h, zbuf)
        pltpu.sync_copy(dst_h.at[pl.ds(c0, _ITERS)], dsti)
        if gather:
            pltpu.sync_copy(src_h.at[pl.ds(c0, _ITERS)], srci)
        else:
            pltpu.sync_copy(ones_h, rows0)

        rbufs = (rows0, rows1)
        gsems = (gsem0, gsem1)
        ssems = (ssem0, ssem1)

        for s in range(nslab):
            def zero_body(j, carry):
                r = _TROWS * t + _ZR * j

                @pl.when(r < _N)
                def _():
                    pltpu.sync_copy(zbuf, acc.at[pl.ds(r, _ZR)])
                return carry
            lax.fori_loop(0, _TROWS // _ZR, zero_body, 0)
            plsc.subcore_barrier()

            if gather:
                def start_gather(j, b):
                    pltpu.async_copy(tbls[s].at[srci.at[j]], rbufs[b],
                                     gsems[b])

                def wait_gather(j, b):
                    pltpu.make_async_copy(tbls[s].at[srci.at[j]], rbufs[b],
                                          gsems[b]).wait()

                def start_scatter(j, b):
                    pltpu.async_copy(rbufs[b], acc.at[dsti.at[j]], ssems[b],
                                     add=True)

                def wait_scatter(j, b):
                    pltpu.make_async_copy(rbufs[b], acc.at[dsti.at[j]],
                                          ssems[b]).wait()

                start_gather(0, 0)
                start_gather(1, 1)

                def edge_body(i, carry):
                    j0 = 2 * i
                    j1 = 2 * i + 1

                    @pl.when(j0 < nv)
                    def _():
                        wait_gather(j0, 0)
                        start_scatter(j0, 0)

                    @pl.when(j1 < nv)
                    def _():
                        wait_gather(j1, 1)
                        start_scatter(j1, 1)

                    @pl.when(j0 + 2 < nv)
                    def _():
                        wait_scatter(j0, 0)
                        start_gather(j0 + 2, 0)

                    @pl.when(j1 + 2 < nv)
                    def _():
                        wait_scatter(j1, 1)
                        start_gather(j1 + 2, 1)
                    return carry
                lax.fori_loop(0, _ITERS // 2, edge_body, 0)
                wait_scatter(0, 0)
                wait_scatter(1, 1)
            else:
                def edge_body(i, carry):
                    @pl.when(i < nv)
                    def _():
                        pltpu.async_copy(rows0, acc.at[dsti.at[i]], ssem0,
                                         add=True)
                    return carry
                lax.fori_loop(0, _ITERS, edge_body, 0)

                def drain_body(i, carry):
                    @pl.when(i < nv)
                    def _():
                        pltpu.make_async_copy(rows0, acc.at[dsti.at[i]],
                                              ssem0).wait()
                    return carry
                lax.fori_loop(0, _ITERS, drain_body, 0)
            plsc.subcore_barrier()

            def out_body(j, carry):
                r = _TROWS * t + _ZR * j

                @pl.when(r < _N)
                def _():
                    pltpu.sync_copy(acc.at[pl.ds(r, _ZR)],
                                    outs[s].at[c, pl.ds(r, _ZR)])
                return carry
            lax.fori_loop(0, _TROWS // _ZR, out_body, 0)
            if s + 1 < nslab:
                plsc.subcore_barrier()

    return pl.kernel(body, out_type=out_type, mesh=mesh,
                     scratch_types=scratch)


def _pad2d(idx):
    return jnp.pad(idx, (0, _ITERS * _NW * _CH - _E)).reshape(-1, _CH)


def _agg_deg(dst2d):
    fill = jnp.zeros((_ZR, 128), jnp.float32)
    ones = jnp.ones((_CH, 128), jnp.float32)
    (degp,) = _sc_agg(128, 1, False)(dst2d, fill, ones)
    return degp


def _agg_rows(src2d, dst2d, tables, D):
    fill = jnp.zeros((_ZR, D), jnp.float32)
    return _sc_agg(D, len(tables), True)(src2d, dst2d, fill, *tables)


_R = 1000


def _tc_scale_x(degp, x):
    def body(degp_ref, x_ref, xp0_ref, xp1_ref, s16_ref):
        deg = degp_ref[0, :, 0:1] + degp_ref[1, :, 0:1] + 1.0
        s = lax.rsqrt(deg)
        xs = x_ref[...] * s
        xp0_ref[...] = xs[:, :128]
        xp1_ref[...] = xs[:, 128:]
        s16_ref[...] = jnp.broadcast_to(s, (_R, 16))

    return pl.pallas_call(
        body,
        grid=(_N // _R,),
        in_specs=[pl.BlockSpec((2, _R, 128), lambda i: (0, i, 0)),
                  pl.BlockSpec((_R, 256), lambda i: (i, 0))],
        out_specs=[pl.BlockSpec((_R, 128), lambda i: (i, 0)),
                   pl.BlockSpec((_R, 128), lambda i: (i, 0)),
                   pl.BlockSpec((_R, 16), lambda i: (i, 0))],
        out_shape=[jax.ShapeDtypeStruct((_N, 128), jnp.float32),
                   jax.ShapeDtypeStruct((_N, 128), jnp.float32),
                   jax.ShapeDtypeStruct((_N, 16), jnp.float32)],
    )(degp, x)


def _tc_mm1(s16, xps, qs, W1, b1):
    def body(s16_ref, xp0, xp1, q0, q1, w_ref, b_ref, o0, o1, o2, o3):
        s = s16_ref[:, 0:1]
        a0 = s * (q0[0] + q0[1] + xp0[...])
        a1 = s * (q1[0] + q1[1] + xp1[...])
        a = jnp.concatenate([a0, a1], axis=1)
        h = jnp.dot(a, w_ref[...], preferred_element_type=jnp.float32)
        h = jnp.maximum(h + b_ref[...], 0.0) * s
        o0[...] = h[:, 0:128]
        o1[...] = h[:, 128:256]
        o2[...] = h[:, 256:384]
        o3[...] = h[:, 384:512]

    slab = pl.BlockSpec((_R, 128), lambda i: (i, 0))
    part = pl.BlockSpec((2, _R, 128), lambda i: (0, i, 0))
    return pl.pallas_call(
        body,
        grid=(_N // _R,),
        in_specs=[pl.BlockSpec((_R, 16), lambda i: (i, 0)),
                  slab, slab, part, part,
                  pl.BlockSpec((256, 512), lambda i: (0, 0)),
                  pl.BlockSpec((1, 512), lambda i: (0, 0))],
        out_specs=[slab, slab, slab, slab],
        out_shape=[jax.ShapeDtypeStruct((_N, 128), jnp.float32)
                   for _ in range(4)],
    )(s16, xps[0], xps[1], qs[0], qs[1], W1, b1.reshape(1, 512))


def _tc_mm23(s16, hps, rs, W2, b2, W3p):
    def body(s16_ref, h0, h1, h2r, h3, r0, r1, r2, r3, w2_ref, b2_ref,
             w3_ref, out_ref):
        s = s16_ref[:, 0:1]
        hs = (h0, h1, h2r, h3)
        rsl = (r0, r1, r2, r3)
        a = jnp.concatenate(
            [s * (rsl[k][0] + rsl[k][1] + hs[k][...]) for k in range(4)],
            axis=1)
        h = jnp.dot(a, w2_ref[...], preferred_element_type=jnp.float32)
        h = jnp.maximum(h + b2_ref[...], 0.0)
        t = jnp.dot(h, w3_ref[...], preferred_element_type=jnp.float32)
        out_ref[...] = t * s

    slab = pl.BlockSpec((_R, 128), lambda i: (i, 0))
    part = pl.BlockSpec((2, _R, 128), lambda i: (0, i, 0))
    return pl.pallas_call(
        body,
        grid=(_N // _R,),
        in_specs=[pl.BlockSpec((_R, 16), lambda i: (i, 0)),
                  slab, slab, slab, slab, part, part, part, part,
                  pl.BlockSpec((512, 512), lambda i: (0, 0)),
                  pl.BlockSpec((1, 512), lambda i: (0, 0)),
                  pl.BlockSpec((512, 128), lambda i: (0, 0))],
        out_specs=pl.BlockSpec((_R, 128), lambda i: (i, 0)),
        out_shape=jax.ShapeDtypeStruct((_N, 128), jnp.float32),
    )(s16, hps[0], hps[1], hps[2], hps[3], rs[0], rs[1], rs[2], rs[3],
      W2, b2.reshape(1, 512), W3p)


def _tc_final(s16, tp, u, b3p):
    def body(s16_ref, tp_ref, u_ref, b_ref, out_ref):
        s = s16_ref[:, 0:1]
        out_ref[...] = s * (u_ref[0] + u_ref[1] + tp_ref[...]) + b_ref[...]

    return pl.pallas_call(
        body,
        grid=(_N // _R,),
        in_specs=[pl.BlockSpec((_R, 16), lambda i: (i, 0)),
                  pl.BlockSpec((_R, 128), lambda i: (i, 0)),
                  pl.BlockSpec((2, _R, 128), lambda i: (0, i, 0)),
                  pl.BlockSpec((1, 128), lambda i: (0, 0))],
        out_specs=pl.BlockSpec((_R, 128), lambda i: (i, 0)),
        out_shape=jax.ShapeDtypeStruct((_N, 128), jnp.float32),
    )(s16, tp, u, b3p)


def kernel(x, edge_index, W1, b1, W2, b2, W3, b3):
    src = _pad2d(edge_index[0])
    dst = _pad2d(edge_index[1])
    W3p = jnp.pad(W3, ((0, 0), (0, 88)))
    b3p = jnp.pad(b3, (0, 88)).reshape(1, 128)

    degp = _agg_deg(dst)
    xp0, xp1, s16 = _tc_scale_x(degp, x)
    qs = _agg_rows(src, dst, [xp0, xp1], 128)
    hps = _tc_mm1(s16, (xp0, xp1), qs, W1, b1)
    rs = _agg_rows(src, dst, hps, 128)
    tp = _tc_mm23(s16, hps, rs, W2, b2, W3p)
    (u,) = _agg_rows(src, dst, [tp], 128)
    out = _tc_final(s16, tp, u, b3p)
    return out[:, :40]


# speedup vs baseline: 1.3028x; 1.3028x over previous
import jax
import jax.numpy as jnp
from jax import lax
from jax.experimental import pallas as pl
from jax.experimental.pallas import tpu as pltpu
from jax.experimental.pallas import tpu_sc as plsc

_N = 10000
_E = 160000
_CH = 128
_NCH = _E // _CH
_NC, _NS = 2, 16
_NW = _NC * _NS
_ITERS = -(-_NCH // _NW)
_ZR = 40
_TROWS = 640


def _sc_agg(nslab, gather, tail, dtype):
    mesh = plsc.VectorSubcoreMesh(core_axis_name="c", subcore_axis_name="s")
    out_type = [jax.ShapeDtypeStruct((_NC, _N) + tail, dtype)
                for _ in range(nslab)]
    scratch = [
        pltpu.VMEM_SHARED((_N,) + tail, dtype),
        pltpu.VMEM((_ZR,) + tail, dtype),
        pltpu.VMEM((_CH,) + tail, dtype),
        pltpu.VMEM((_CH,) + tail, dtype),
        pltpu.VMEM((_ITERS, _CH), jnp.int32),
        pltpu.VMEM((_ITERS, _CH), jnp.int32),
        pltpu.SemaphoreType.DMA,
        pltpu.SemaphoreType.DMA,
    ]

    def body(*refs):
        n_in = (3 if gather else 2) + nslab
        ins, outs = refs[:n_in], refs[n_in:n_in + nslab]
        acc, zbuf, rows0, rows1, srci, dsti, gsem0, gsem1 = refs[n_in + nslab:]
        if gather:
            src_h, dst_h, fill_h = ins[0], ins[1], ins[2]
            tbls = ins[3:]
        else:
            dst_h, fill_h, ones_h = ins[0], ins[1], ins[2]
            tbls = ()

        c = lax.axis_index("c")
        t = lax.axis_index("s")
        w = t * _NC + c
        c0 = _ITERS * w
        nv = lax.min(_ITERS, _NCH - c0)

        pltpu.sync_copy(fill_h, zbuf)
        pltpu.sync_copy(dst_h.at[pl.ds(c0, _ITERS)], dsti)
        if gather:
            pltpu.sync_copy(src_h.at[pl.ds(c0, _ITERS)], srci)
        else:
            pltpu.sync_copy(ones_h, rows0)

        rbufs = (rows0, rows1)
        gsems = (gsem0, gsem1)

        for s in range(nslab):
            def zero_body(j, carry):
                r = _TROWS * t + _ZR * j

                @pl.when(r < _N)
                def _():
                    pltpu.async_copy(zbuf, acc.at[pl.ds(r, _ZR)], gsem0)
                return carry
            lax.fori_loop(0, _TROWS // _ZR, zero_body, 0)

            def zero_drain(j, carry):
                r = _TROWS * t + _ZR * j

                @pl.when(r < _N)
                def _():
                    pltpu.make_async_copy(zbuf, acc.at[pl.ds(r, _ZR)],
                                          gsem0).wait()
                return carry
            lax.fori_loop(0, _TROWS // _ZR, zero_drain, 0)
            plsc.subcore_barrier()

            if gather:
                def start(j, b):
                    pltpu.async_copy(tbls[s].at[srci.at[j]], rbufs[b],
                                     gsems[b])

                def finish(j, b):
                    pltpu.make_async_copy(tbls[s].at[srci.at[j]], rbufs[b],
                                          gsems[b]).wait()
                    pltpu.sync_copy(rbufs[b], acc.at[dsti.at[j]], add=True)

                start(0, 0)

                def edge_body(i, carry):
                    j0 = 2 * i
                    j1 = 2 * i + 1
                    j2 = 2 * i + 2

                    @pl.when(j1 < nv)
                    def _():
                        start(j1, 1)

                    @pl.when(j0 < nv)
                    def _():
                        finish(j0, 0)

                    @pl.when(j2 < nv)
                    def _():
                        start(j2, 0)

                    @pl.when(j1 < nv)
                    def _():
                        finish(j1, 1)
                    return carry
                lax.fori_loop(0, _ITERS // 2, edge_body, 0)
            else:
                def edge_body(i, carry):
                    @pl.when(i < nv)
                    def _():
                        pltpu.sync_copy(rows0, acc.at[dsti.at[i]], add=True)
                    return carry
                lax.fori_loop(0, _ITERS, edge_body, 0)
            plsc.subcore_barrier()

            def out_body(j, carry):
                r = _TROWS * t + _ZR * j

                @pl.when(r < _N)
                def _():
                    pltpu.async_copy(acc.at[pl.ds(r, _ZR)],
                                     outs[s].at[c, pl.ds(r, _ZR)], gsem0)
                return carry
            lax.fori_loop(0, _TROWS // _ZR, out_body, 0)

            def out_drain(j, carry):
                r = _TROWS * t + _ZR * j

                @pl.when(r < _N)
                def _():
                    pltpu.make_async_copy(acc.at[pl.ds(r, _ZR)],
                                          outs[s].at[c, pl.ds(r, _ZR)],
                                          gsem0).wait()
                return carry
            lax.fori_loop(0, _TROWS // _ZR, out_drain, 0)
            if s + 1 < nslab:
                plsc.subcore_barrier()

    return pl.kernel(body, out_type=out_type, mesh=mesh,
                     scratch_types=scratch)


def _pad2d(idx):
    return jnp.pad(idx, (0, _ITERS * _NW * _CH - _E)).reshape(-1, _CH)


def _agg_deg(dst2d):
    fill = jnp.zeros((_ZR, 128), jnp.float32)
    ones = jnp.ones((_CH, 128), jnp.float32)
    (degp,) = _sc_agg(1, False, (128,), jnp.float32)(dst2d, fill, ones)
    return degp


def _agg_f32(src2d, dst2d, tables):
    fill = jnp.zeros((_ZR, 128), jnp.float32)
    return _sc_agg(len(tables), True, (128,), jnp.float32)(
        src2d, dst2d, fill, *tables)


_R = 1000


def _tc_scale_x(degp, x):
    def body(degp_ref, x_ref, xp0_ref, xp1_ref, s16_ref):
        deg = degp_ref[0, :, 0:1] + degp_ref[1, :, 0:1] + 1.0
        s = lax.rsqrt(deg)
        xs = x_ref[...] * s
        xp0_ref[...] = xs[:, :128]
        xp1_ref[...] = xs[:, 128:]
        s16_ref[...] = jnp.broadcast_to(s, (_R, 16))

    return pl.pallas_call(
        body,
        grid=(_N // _R,),
        in_specs=[pl.BlockSpec((2, _R, 128), lambda i: (0, i, 0)),
                  pl.BlockSpec((_R, 256), lambda i: (i, 0))],
        out_specs=[pl.BlockSpec((_R, 128), lambda i: (i, 0)),
                   pl.BlockSpec((_R, 128), lambda i: (i, 0)),
                   pl.BlockSpec((_R, 16), lambda i: (i, 0))],
        out_shape=[jax.ShapeDtypeStruct((_N, 128), jnp.float32),
                   jax.ShapeDtypeStruct((_N, 128), jnp.float32),
                   jax.ShapeDtypeStruct((_N, 16), jnp.float32)],
    )(degp, x)


def _tc_mm1(s16, xps, qs, W1, b1):
    def body(s16_ref, xp0, xp1, q0, q1, w_ref, b_ref, o0, o1, o2, o3):
        s = s16_ref[:, 0:1]
        a0 = s * (q0[0] + q0[1] + xp0[...])
        a1 = s * (q1[0] + q1[1] + xp1[...])
        a = jnp.concatenate([a0, a1], axis=1)
        h = jnp.dot(a, w_ref[...], preferred_element_type=jnp.float32)
        h = jnp.maximum(h + b_ref[...], 0.0) * s
        o0[...] = h[:, 0:128]
        o1[...] = h[:, 128:256]
        o2[...] = h[:, 256:384]
        o3[...] = h[:, 384:512]

    slab = pl.BlockSpec((_R, 128), lambda i: (i, 0))
    part = pl.BlockSpec((2, _R, 128), lambda i: (0, i, 0))
    return pl.pallas_call(
        body,
        grid=(_N // _R,),
        in_specs=[pl.BlockSpec((_R, 16), lambda i: (i, 0)),
                  slab, slab, part, part,
                  pl.BlockSpec((256, 512), lambda i: (0, 0)),
                  pl.BlockSpec((1, 512), lambda i: (0, 0))],
        out_specs=[slab, slab, slab, slab],
        out_shape=[jax.ShapeDtypeStruct((_N, 128), jnp.float32)
                   for _ in range(4)],
    )(s16, xps[0], xps[1], qs[0], qs[1], W1, b1.reshape(1, 512))


def _tc_mm23(s16, hps, rs, W2, b2, W3p):
    def body(s16_ref, h0, h1, h2r, h3, r0, r1, r2, r3, w2_ref, b2_ref,
             w3_ref, out_ref):
        s = s16_ref[:, 0:1]
        hs = (h0, h1, h2r, h3)
        rsl = (r0, r1, r2, r3)
        a = jnp.concatenate(
            [s * (rsl[k][0] + rsl[k][1] + hs[k][...]) for k in range(4)],
            axis=1)
        h = jnp.dot(a, w2_ref[...], preferred_element_type=jnp.float32)
        h = jnp.maximum(h + b2_ref[...], 0.0)
        t = jnp.dot(h, w3_ref[...], preferred_element_type=jnp.float32)
        out_ref[...] = t * s

    slab = pl.BlockSpec((_R, 128), lambda i: (i, 0))
    part = pl.BlockSpec((2, _R, 128), lambda i: (0, i, 0))
    return pl.pallas_call(
        body,
        grid=(_N // _R,),
        in_specs=[pl.BlockSpec((_R, 16), lambda i: (i, 0)),
                  slab, slab, slab, slab, part, part, part, part,
                  pl.BlockSpec((512, 512), lambda i: (0, 0)),
                  pl.BlockSpec((1, 512), lambda i: (0, 0)),
                  pl.BlockSpec((512, 128), lambda i: (0, 0))],
        out_specs=pl.BlockSpec((_R, 128), lambda i: (i, 0)),
        out_shape=jax.ShapeDtypeStruct((_N, 128), jnp.float32),
    )(s16, hps[0], hps[1], hps[2], hps[3], rs[0], rs[1], rs[2], rs[3],
      W2, b2.reshape(1, 512), W3p)


def _tc_final(s16, tp, u, b3p):
    def body(s16_ref, tp_ref, u_ref, b_ref, out_ref):
        s = s16_ref[:, 0:1]
        out_ref[...] = s * (u_ref[0] + u_ref[1] + tp_ref[...]) + b_ref[...]

    return pl.pallas_call(
        body,
        grid=(_N // _R,),
        in_specs=[pl.BlockSpec((_R, 16), lambda i: (i, 0)),
                  pl.BlockSpec((_R, 128), lambda i: (i, 0)),
                  pl.BlockSpec((2, _R, 128), lambda i: (0, i, 0)),
                  pl.BlockSpec((1, 128), lambda i: (0, 0))],
        out_specs=pl.BlockSpec((_R, 128), lambda i: (i, 0)),
        out_shape=jax.ShapeDtypeStruct((_N, 128), jnp.float32),
    )(s16, tp, u, b3p)


def kernel(x, edge_index, W1, b1, W2, b2, W3, b3):
    src = _pad2d(edge_index[0])
    dst = _pad2d(edge_index[1])
    W3p = jnp.pad(W3, ((0, 0), (0, 88)))
    b3p = jnp.pad(b3, (0, 88)).reshape(1, 128)

    degp = _agg_deg(dst)
    xp0, xp1, s16 = _tc_scale_x(degp, x)
    qs = _agg_f32(src, dst, [xp0, xp1])
    hps = _tc_mm1(s16, (xp0, xp1), qs, W1, b1)
    rs = _agg_f32(src, dst, hps)
    tp = _tc_mm23(s16, hps, rs, W2, b2, W3p)
    (u,) = _agg_f32(src, dst, [tp])
    out = _tc_final(s16, tp, u, b3p)
    return out[:, :40]


# speedup vs baseline: 1.3034x; 1.0005x over previous
import jax
import jax.numpy as jnp
from jax import lax
from jax.experimental import pallas as pl
from jax.experimental.pallas import tpu as pltpu
from jax.experimental.pallas import tpu_sc as plsc

_N = 10000
_E = 160000
_CH = 128
_NCH = _E // _CH
_NC, _NS = 2, 16
_NW = _NC * _NS
_ITERS = -(-_NCH // _NW)
_ZR = 40
_TROWS = 640


def _sc_agg(nslab, gather, tail, dtype):
    mesh = plsc.VectorSubcoreMesh(core_axis_name="c", subcore_axis_name="s")
    out_type = [jax.ShapeDtypeStruct((_NC, _N) + tail, dtype)
                for _ in range(nslab)]
    scratch = [
        pltpu.VMEM_SHARED((_N,) + tail, dtype),
        pltpu.VMEM((_ZR,) + tail, dtype),
        pltpu.VMEM((_CH,) + tail, dtype),
        pltpu.VMEM((_CH,) + tail, dtype),
        pltpu.VMEM((_ITERS, _CH), jnp.int32),
        pltpu.VMEM((_ITERS, _CH), jnp.int32),
        pltpu.SemaphoreType.DMA,
        pltpu.SemaphoreType.DMA,
    ]

    def body(*refs):
        n_in = (3 if gather else 2) + nslab
        ins, outs = refs[:n_in], refs[n_in:n_in + nslab]
        acc, zbuf, rows0, rows1, srci, dsti, gsem0, gsem1 = refs[n_in + nslab:]
        if gather:
            src_h, dst_h, fill_h = ins[0], ins[1], ins[2]
            tbls = ins[3:]
        else:
            dst_h, fill_h, ones_h = ins[0], ins[1], ins[2]
            tbls = ()

        c = lax.axis_index("c")
        t = lax.axis_index("s")
        w = t * _NC + c
        c0 = _ITERS * w
        nv = lax.min(_ITERS, _NCH - c0)

        pltpu.sync_copy(fill_h, zbuf)
        pltpu.sync_copy(dst_h.at[pl.ds(c0, _ITERS)], dsti)
        if gather:
            pltpu.sync_copy(src_h.at[pl.ds(c0, _ITERS)], srci)
        else:
            pltpu.sync_copy(ones_h, rows0)

        rbufs = (rows0, rows1)
        gsems = (gsem0, gsem1)

        for s in range(nslab):
            def zero_body(j, carry):
                r = _TROWS * t + _ZR * j

                @pl.when(r < _N)
                def _():
                    pltpu.async_copy(zbuf, acc.at[pl.ds(r, _ZR)], gsem0)
                return carry
            lax.fori_loop(0, _TROWS // _ZR, zero_body, 0)

            def zero_drain(j, carry):
                r = _TROWS * t + _ZR * j

                @pl.when(r < _N)
                def _():
                    pltpu.make_async_copy(zbuf, acc.at[pl.ds(r, _ZR)],
                                          gsem0).wait()
                return carry
            lax.fori_loop(0, _TROWS // _ZR, zero_drain, 0)
            plsc.subcore_barrier()

            if gather:
                def start(j, b):
                    pltpu.async_copy(tbls[s].at[srci.at[j]], rbufs[b],
                                     gsems[b])

                def finish(j, b):
                    pltpu.make_async_copy(tbls[s].at[srci.at[j]], rbufs[b],
                                          gsems[b]).wait()
                    pltpu.sync_copy(rbufs[b], acc.at[dsti.at[j]], add=True)

                start(0, 0)

                def edge_body(i, carry):
                    j0 = 2 * i
                    j1 = 2 * i + 1
                    j2 = 2 * i + 2

                    @pl.when(j1 < nv)
                    def _():
                        start(j1, 1)

                    @pl.when(j0 < nv)
                    def _():
                        finish(j0, 0)

                    @pl.when(j2 < nv)
                    def _():
                        start(j2, 0)

                    @pl.when(j1 < nv)
                    def _():
                        finish(j1, 1)
                    return carry
                lax.fori_loop(0, _ITERS // 2, edge_body, 0)
            else:
                def edge_body(i, carry):
                    @pl.when(i < nv)
                    def _():
                        pltpu.sync_copy(rows0, acc.at[dsti.at[i]], add=True)
                    return carry
                lax.fori_loop(0, _ITERS, edge_body, 0)
            plsc.subcore_barrier()

            def out_body(j, carry):
                r = _TROWS * t + _ZR * j

                @pl.when(r < _N)
                def _():
                    pltpu.async_copy(acc.at[pl.ds(r, _ZR)],
                                     outs[s].at[c, pl.ds(r, _ZR)], gsem0)
                return carry
            lax.fori_loop(0, _TROWS // _ZR, out_body, 0)

            def out_drain(j, carry):
                r = _TROWS * t + _ZR * j

                @pl.when(r < _N)
                def _():
                    pltpu.make_async_copy(acc.at[pl.ds(r, _ZR)],
                                          outs[s].at[c, pl.ds(r, _ZR)],
                                          gsem0).wait()
                return carry
            lax.fori_loop(0, _TROWS // _ZR, out_drain, 0)

    return pl.kernel(body, out_type=out_type, mesh=mesh,
                     scratch_types=scratch)


def _pad2d(idx):
    return jnp.pad(idx, (0, _ITERS * _NW * _CH - _E)).reshape(-1, _CH)


def _agg_deg(dst2d):
    fill = jnp.zeros((_ZR, 128), jnp.float32)
    ones = jnp.ones((_CH, 128), jnp.float32)
    (degp,) = _sc_agg(1, False, (128,), jnp.float32)(dst2d, fill, ones)
    return degp


def _agg_f32(src2d, dst2d, tables):
    fill = jnp.zeros((_ZR, 128), jnp.float32)
    return _sc_agg(len(tables), True, (128,), jnp.float32)(
        src2d, dst2d, fill, *tables)


_R = 1000


def _tc_scale_x(degp, x):
    def body(degp_ref, x_ref, xp0_ref, xp1_ref, s16_ref):
        deg = degp_ref[0, :, 0:1] + degp_ref[1, :, 0:1] + 1.0
        s = lax.rsqrt(deg)
        xs = x_ref[...] * s
        xp0_ref[...] = xs[:, :128]
        xp1_ref[...] = xs[:, 128:]
        s16_ref[...] = jnp.broadcast_to(s, (_R, 16))

    return pl.pallas_call(
        body,
        grid=(_N // _R,),
        in_specs=[pl.BlockSpec((2, _R, 128), lambda i: (0, i, 0)),
                  pl.BlockSpec((_R, 256), lambda i: (i, 0))],
        out_specs=[pl.BlockSpec((_R, 128), lambda i: (i, 0)),
                   pl.BlockSpec((_R, 128), lambda i: (i, 0)),
                   pl.BlockSpec((_R, 16), lambda i: (i, 0))],
        out_shape=[jax.ShapeDtypeStruct((_N, 128), jnp.float32),
                   jax.ShapeDtypeStruct((_N, 128), jnp.float32),
                   jax.ShapeDtypeStruct((_N, 16), jnp.float32)],
    )(degp, x)


def _tc_mm1(s16, xps, qs, W1, b1):
    def body(s16_ref, xp0, xp1, q0, q1, w_ref, b_ref, o0, o1, o2, o3):
        s = s16_ref[:, 0:1]
        a0 = s * (q0[0] + q0[1] + xp0[...])
        a1 = s * (q1[0] + q1[1] + xp1[...])
        a = jnp.concatenate([a0, a1], axis=1)
        h = jnp.dot(a, w_ref[...], preferred_element_type=jnp.float32)
        h = jnp.maximum(h + b_ref[...], 0.0) * s
        o0[...] = h[:, 0:128]
        o1[...] = h[:, 128:256]
        o2[...] = h[:, 256:384]
        o3[...] = h[:, 384:512]

    slab = pl.BlockSpec((_R, 128), lambda i: (i, 0))
    part = pl.BlockSpec((2, _R, 128), lambda i: (0, i, 0))
    return pl.pallas_call(
        body,
        grid=(_N // _R,),
        in_specs=[pl.BlockSpec((_R, 16), lambda i: (i, 0)),
                  slab, slab, part, part,
                  pl.BlockSpec((256, 512), lambda i: (0, 0)),
                  pl.BlockSpec((1, 512), lambda i: (0, 0))],
        out_specs=[slab, slab, slab, slab],
        out_shape=[jax.ShapeDtypeStruct((_N, 128), jnp.float32)
                   for _ in range(4)],
    )(s16, xps[0], xps[1], qs[0], qs[1], W1, b1.reshape(1, 512))


def _tc_mm23(s16, hps, rs, W2, b2, W3p):
    def body(s16_ref, h0, h1, h2r, h3, r0, r1, r2, r3, w2_ref, b2_ref,
             w3_ref, out_ref):
        s = s16_ref[:, 0:1]
        hs = (h0, h1, h2r, h3)
        rsl = (r0, r1, r2, r3)
        a = jnp.concatenate(
            [s * (rsl[k][0] + rsl[k][1] + hs[k][...]) for k in range(4)],
            axis=1)
        h = jnp.dot(a, w2_ref[...], preferred_element_type=jnp.float32)
        h = jnp.maximum(h + b2_ref[...], 0.0)
        t = jnp.dot(h, w3_ref[...], preferred_element_type=jnp.float32)
        out_ref[...] = t * s

    slab = pl.BlockSpec((_R, 128), lambda i: (i, 0))
    part = pl.BlockSpec((2, _R, 128), lambda i: (0, i, 0))
    return pl.pallas_call(
        body,
        grid=(_N // _R,),
        in_specs=[pl.BlockSpec((_R, 16), lambda i: (i, 0)),
                  slab, slab, slab, slab, part, part, part, part,
                  pl.BlockSpec((512, 512), lambda i: (0, 0)),
                  pl.BlockSpec((1, 512), lambda i: (0, 0)),
                  pl.BlockSpec((512, 128), lambda i: (0, 0))],
        out_specs=pl.BlockSpec((_R, 128), lambda i: (i, 0)),
        out_shape=jax.ShapeDtypeStruct((_N, 128), jnp.float32),
    )(s16, hps[0], hps[1], hps[2], hps[3], rs[0], rs[1], rs[2], rs[3],
      W2, b2.reshape(1, 512), W3p)


def _tc_final(s16, tp, u, b3p):
    def body(s16_ref, tp_ref, u_ref, b_ref, out_ref):
        s = s16_ref[:, 0:1]
        out_ref[...] = s * (u_ref[0] + u_ref[1] + tp_ref[...]) + b_ref[...]

    return pl.pallas_call(
        body,
        grid=(_N // _R,),
        in_specs=[pl.BlockSpec((_R, 16), lambda i: (i, 0)),
                  pl.BlockSpec((_R, 128), lambda i: (i, 0)),
                  pl.BlockSpec((2, _R, 128), lambda i: (0, i, 0)),
                  pl.BlockSpec((1, 128), lambda i: (0, 0))],
        out_specs=pl.BlockSpec((_R, 128), lambda i: (i, 0)),
        out_shape=jax.ShapeDtypeStruct((_N, 128), jnp.float32),
    )(s16, tp, u, b3p)


def kernel(x, edge_index, W1, b1, W2, b2, W3, b3):
    src = _pad2d(edge_index[0])
    dst = _pad2d(edge_index[1])
    W3p = jnp.pad(W3, ((0, 0), (0, 88)))
    b3p = jnp.pad(b3, (0, 88)).reshape(1, 128)

    degp = _agg_deg(dst)
    xp0, xp1, s16 = _tc_scale_x(degp, x)
    qs = _agg_f32(src, dst, [xp0, xp1])
    hps = _tc_mm1(s16, (xp0, xp1), qs, W1, b1)
    rs = _agg_f32(src, dst, hps)
    tp = _tc_mm23(s16, hps, rs, W2, b2, W3p)
    (u,) = _agg_f32(src, dst, [tp])
    out = _tc_final(s16, tp, u, b3p)
    return out[:, :40]


# speedup vs baseline: 1.3502x; 1.0360x over previous
import jax
import jax.numpy as jnp
from jax import lax
from jax.experimental import pallas as pl
from jax.experimental.pallas import tpu as pltpu
from jax.experimental.pallas import tpu_sc as plsc

_N = 10000
_E = 160000
_CH = 128
_NCH = _E // _CH
_NC, _NS = 2, 16
_NW = _NC * _NS
_ITERS = -(-_NCH // _NW)
_ZR = 40
_TROWS = 640


def _sc_agg(nslab, gather, tail, dtype):
    mesh = plsc.VectorSubcoreMesh(core_axis_name="c", subcore_axis_name="s")
    out_type = [jax.ShapeDtypeStruct((_NC, _N) + tail, dtype)
                for _ in range(nslab)]
    scratch = [
        pltpu.VMEM_SHARED((_N,) + tail, dtype),
        pltpu.VMEM((_ZR,) + tail, dtype),
        pltpu.VMEM((_CH,) + tail, dtype),
        pltpu.VMEM((_CH,) + tail, dtype),
        pltpu.VMEM((_ITERS, _CH), jnp.int32),
        pltpu.VMEM((_ITERS, _CH), jnp.int32),
        pltpu.SemaphoreType.DMA,
        pltpu.SemaphoreType.DMA,
    ]

    def body(*refs):
        n_in = (3 if gather else 2) + nslab
        ins, outs = refs[:n_in], refs[n_in:n_in + nslab]
        acc, zbuf, rows0, rows1, srci, dsti, gsem0, gsem1 = refs[n_in + nslab:]
        if gather:
            src_h, dst_h, fill_h = ins[0], ins[1], ins[2]
            tbls = ins[3:]
        else:
            dst_h, fill_h, ones_h = ins[0], ins[1], ins[2]
            tbls = ()

        c = lax.axis_index("c")
        t = lax.axis_index("s")
        w = t * _NC + c
        c0 = _ITERS * w
        nv = lax.min(_ITERS, _NCH - c0)

        pltpu.sync_copy(fill_h, zbuf)
        pltpu.sync_copy(dst_h.at[pl.ds(c0, _ITERS)], dsti)
        if gather:
            pltpu.sync_copy(src_h.at[pl.ds(c0, _ITERS)], srci)
        else:
            pltpu.sync_copy(ones_h, rows0)

        rbufs = (rows0, rows1)
        gsems = (gsem0, gsem1)

        for s in range(nslab):
            def zero_body(j, carry):
                r = _TROWS * t + _ZR * j

                @pl.when(r < _N)
                def _():
                    pltpu.async_copy(zbuf, acc.at[pl.ds(r, _ZR)], gsem0)
                return carry
            lax.fori_loop(0, _TROWS // _ZR, zero_body, 0)

            def zero_drain(j, carry):
                r = _TROWS * t + _ZR * j

                @pl.when(r < _N)
                def _():
                    pltpu.make_async_copy(zbuf, acc.at[pl.ds(r, _ZR)],
                                          gsem0).wait()
                return carry
            lax.fori_loop(0, _TROWS // _ZR, zero_drain, 0)
            plsc.subcore_barrier()

            if gather:
                def start(j, b, tb=s):
                    pltpu.async_copy(tbls[tb].at[srci.at[j]], rbufs[b],
                                     gsems[b])

                def finish(j, b):
                    pltpu.make_async_copy(tbls[s].at[srci.at[j]], rbufs[b],
                                          gsems[b]).wait()
                    pltpu.sync_copy(rbufs[b], acc.at[dsti.at[j]], add=True)

                if s == 0:
                    start(0, 0)

                def edge_body(i, carry):
                    j0 = 2 * i
                    j1 = 2 * i + 1
                    j2 = 2 * i + 2

                    @pl.when(j1 < nv)
                    def _():
                        start(j1, 1)

                    @pl.when(j0 < nv)
                    def _():
                        finish(j0, 0)

                    @pl.when(j2 < nv)
                    def _():
                        start(j2, 0)

                    @pl.when(j1 < nv)
                    def _():
                        finish(j1, 1)
                    return carry
                lax.fori_loop(0, _ITERS // 2, edge_body, 0)
                if s + 1 < nslab:
                    start(0, 0, tb=s + 1)
            else:
                def edge_body(i, carry):
                    @pl.when(i < nv)
                    def _():
                        pltpu.sync_copy(rows0, acc.at[dsti.at[i]], add=True)
                    return carry
                lax.fori_loop(0, _ITERS, edge_body, 0)
            plsc.subcore_barrier()

            def out_body(j, carry):
                r = _TROWS * t + _ZR * j

                @pl.when(r < _N)
                def _():
                    pltpu.async_copy(acc.at[pl.ds(r, _ZR)],
                                     outs[s].at[c, pl.ds(r, _ZR)], gsem0)
                return carry
            lax.fori_loop(0, _TROWS // _ZR, out_body, 0)

            def out_drain(j, carry):
                r = _TROWS * t + _ZR * j

                @pl.when(r < _N)
                def _():
                    pltpu.make_async_copy(acc.at[pl.ds(r, _ZR)],
                                          outs[s].at[c, pl.ds(r, _ZR)],
                                          gsem0).wait()
                return carry
            lax.fori_loop(0, _TROWS // _ZR, out_drain, 0)

    return pl.kernel(body, out_type=out_type, mesh=mesh,
                     scratch_types=scratch)


def _pad2d(idx):
    return jnp.pad(idx, (0, _ITERS * _NW * _CH - _E)).reshape(-1, _CH)


def _agg_deg(dst2d):
    fill = jnp.zeros((_ZR, 128), jnp.float32)
    ones = jnp.ones((_CH, 128), jnp.float32)
    (degp,) = _sc_agg(1, False, (128,), jnp.float32)(dst2d, fill, ones)
    return degp


def _agg_f32(src2d, dst2d, tables):
    fill = jnp.zeros((_ZR, 128), jnp.float32)
    return _sc_agg(len(tables), True, (128,), jnp.float32)(
        src2d, dst2d, fill, *tables)


_R = 1000


def _tc_scale_x(degp, x):
    def body(degp_ref, x_ref, xp0_ref, xp1_ref, s16_ref):
        deg = degp_ref[0, :, 0:1] + degp_ref[1, :, 0:1] + 1.0
        s = lax.rsqrt(deg)
        xs = x_ref[...] * s
        xp0_ref[...] = xs[:, :128]
        xp1_ref[...] = xs[:, 128:]
        s16_ref[...] = jnp.broadcast_to(s, (_R, 16))

    return pl.pallas_call(
        body,
        grid=(_N // _R,),
        in_specs=[pl.BlockSpec((2, _R, 128), lambda i: (0, i, 0)),
                  pl.BlockSpec((_R, 256), lambda i: (i, 0))],
        out_specs=[pl.BlockSpec((_R, 128), lambda i: (i, 0)),
                   pl.BlockSpec((_R, 128), lambda i: (i, 0)),
                   pl.BlockSpec((_R, 16), lambda i: (i, 0))],
        out_shape=[jax.ShapeDtypeStruct((_N, 128), jnp.float32),
                   jax.ShapeDtypeStruct((_N, 128), jnp.float32),
                   jax.ShapeDtypeStruct((_N, 16), jnp.float32)],
    )(degp, x)


def _tc_mm1(s16, xps, qs, W1, b1):
    def body(s16_ref, xp0, xp1, q0, q1, w_ref, b_ref, o0, o1, o2, o3):
        s = s16_ref[:, 0:1]
        a0 = s * (q0[0] + q0[1] + xp0[...])
        a1 = s * (q1[0] + q1[1] + xp1[...])
        a = jnp.concatenate([a0, a1], axis=1)
        h = jnp.dot(a, w_ref[...], preferred_element_type=jnp.float32)
        h = jnp.maximum(h + b_ref[...], 0.0) * s
        o0[...] = h[:, 0:128]
        o1[...] = h[:, 128:256]
        o2[...] = h[:, 256:384]
        o3[...] = h[:, 384:512]

    slab = pl.BlockSpec((_R, 128), lambda i: (i, 0))
    part = pl.BlockSpec((2, _R, 128), lambda i: (0, i, 0))
    return pl.pallas_call(
        body,
        grid=(_N // _R,),
        in_specs=[pl.BlockSpec((_R, 16), lambda i: (i, 0)),
                  slab, slab, part, part,
                  pl.BlockSpec((256, 512), lambda i: (0, 0)),
                  pl.BlockSpec((1, 512), lambda i: (0, 0))],
        out_specs=[slab, slab, slab, slab],
        out_shape=[jax.ShapeDtypeStruct((_N, 128), jnp.float32)
                   for _ in range(4)],
    )(s16, xps[0], xps[1], qs[0], qs[1], W1, b1.reshape(1, 512))


def _tc_mm23(s16, hps, rs, W2, b2, W3p):
    def body(s16_ref, h0, h1, h2r, h3, r0, r1, r2, r3, w2_ref, b2_ref,
             w3_ref, out_ref):
        s = s16_ref[:, 0:1]
        hs = (h0, h1, h2r, h3)
        rsl = (r0, r1, r2, r3)
        a = jnp.concatenate(
            [s * (rsl[k][0] + rsl[k][1] + hs[k][...]) for k in range(4)],
            axis=1)
        h = jnp.dot(a, w2_ref[...], preferred_element_type=jnp.float32)
        h = jnp.maximum(h + b2_ref[...], 0.0)
        t = jnp.dot(h, w3_ref[...], preferred_element_type=jnp.float32)
        out_ref[...] = t * s

    slab = pl.BlockSpec((_R, 128), lambda i: (i, 0))
    part = pl.BlockSpec((2, _R, 128), lambda i: (0, i, 0))
    return pl.pallas_call(
        body,
        grid=(_N // _R,),
        in_specs=[pl.BlockSpec((_R, 16), lambda i: (i, 0)),
                  slab, slab, slab, slab, part, part, part, part,
                  pl.BlockSpec((512, 512), lambda i: (0, 0)),
                  pl.BlockSpec((1, 512), lambda i: (0, 0)),
                  pl.BlockSpec((512, 128), lambda i: (0, 0))],
        out_specs=pl.BlockSpec((_R, 128), lambda i: (i, 0)),
        out_shape=jax.ShapeDtypeStruct((_N, 128), jnp.float32),
    )(s16, hps[0], hps[1], hps[2], hps[3], rs[0], rs[1], rs[2], rs[3],
      W2, b2.reshape(1, 512), W3p)


def _tc_final(s16, tp, u, b3p):
    def body(s16_ref, tp_ref, u_ref, b_ref, out_ref):
        s = s16_ref[:, 0:1]
        out_ref[...] = s * (u_ref[0] + u_ref[1] + tp_ref[...]) + b_ref[...]

    return pl.pallas_call(
        body,
        grid=(_N // _R,),
        in_specs=[pl.BlockSpec((_R, 16), lambda i: (i, 0)),
                  pl.BlockSpec((_R, 128), lambda i: (i, 0)),
                  pl.BlockSpec((2, _R, 128), lambda i: (0, i, 0)),
                  pl.BlockSpec((1, 128), lambda i: (0, 0))],
        out_specs=pl.BlockSpec((_R, 128), lambda i: (i, 0)),
        out_shape=jax.ShapeDtypeStruct((_N, 128), jnp.float32),
    )(s16, tp, u, b3p)


def kernel(x, edge_index, W1, b1, W2, b2, W3, b3):
    src = _pad2d(edge_index[0])
    dst = _pad2d(edge_index[1])
    W3p = jnp.pad(W3, ((0, 0), (0, 88)))
    b3p = jnp.pad(b3, (0, 88)).reshape(1, 128)

    degp = _agg_deg(dst)
    xp0, xp1, s16 = _tc_scale_x(degp, x)
    qs = _agg_f32(src, dst, [xp0, xp1])
    hps = _tc_mm1(s16, (xp0, xp1), qs, W1, b1)
    rs = _agg_f32(src, dst, hps)
    tp = _tc_mm23(s16, hps, rs, W2, b2, W3p)
    (u,) = _agg_f32(src, dst, [tp])
    out = _tc_final(s16, tp, u, b3p)
    return out[:, :40]
